# Initial kernel scaffold; baseline (speedup 1.0000x reference)
import jax
import jax.numpy as jnp
from jax.experimental import pallas as pl


def kernel(x, track_ids, time_pe, track_pe):
    raise NotImplementedError("write your pallas kernel here")


# speedup vs baseline: 1.0889x; 1.0889x over previous
import jax
import jax.numpy as jnp
from jax import lax
from jax.experimental import pallas as pl
from jax.experimental.pallas import tpu as pltpu

R_BLK = 512


def _body(x_ref, ids_ref, tpe_ref, trk_ref, o_ref):
    x = x_ref[...]
    ids = ids_ref[...]
    t = tpe_ref[...]
    trk = trk_ref[...]
    n = x.shape[0]
    iota8 = lax.broadcasted_iota(jnp.int32, (n, 8), 1)
    onehot = (ids == iota8).astype(jnp.float32)
    row_i = lax.broadcasted_iota(jnp.int32, (n, n // 4), 0)
    col_i = lax.broadcasted_iota(jnp.int32, (n, n // 4), 1)
    expand = (row_i // 4 == col_i).astype(jnp.float32)
    enc = jnp.dot(onehot, trk, preferred_element_type=jnp.float32)
    t_exp = jnp.dot(expand, t, preferred_element_type=jnp.float32)
    o_ref[...] = x + t_exp + enc


@jax.jit
def kernel(x, track_ids, time_pe, track_pe):
    S, B, D = x.shape
    R = S * B
    x2 = x.reshape(R, D)
    ids2 = track_ids.reshape(R, 1)
    grid = (R // R_BLK,)
    out = pl.pallas_call(
        _body,
        grid=grid,
        in_specs=[
            pl.BlockSpec((R_BLK, D), lambda i: (i, 0)),
            pl.BlockSpec((R_BLK, 1), lambda i: (i, 0)),
            pl.BlockSpec((R_BLK // 4, D), lambda i: (i, 0)),
            pl.BlockSpec((8, D), lambda i: (0, 0)),
        ],
        out_specs=pl.BlockSpec((R_BLK, D), lambda i: (i, 0)),
        out_shape=jax.ShapeDtypeStruct((R, D), x.dtype),
        compiler_params=pltpu.CompilerParams(
            dimension_semantics=("arbitrary",),
        ),
    )(x2, ids2, time_pe[:S], track_pe)
    return out.reshape(S, B, D)


# speedup vs baseline: 2.4882x; 2.2851x over previous
import jax
import jax.numpy as jnp
from jax.experimental import pallas as pl
from jax.experimental.pallas import tpu as pltpu

S_BLK = 256


def _body(x_ref, ids_ref, tpe_ref, trk_ref, o_ref):
    x = x_ref[...]
    ids = ids_ref[...]
    t = tpe_ref[...]
    acc = x + t[:, None, :]
    enc = jnp.broadcast_to(trk_ref[0:1, :][None, :, :], x.shape)
    for k in range(1, 8):
        enc = jnp.where(ids == k, trk_ref[k:k + 1, :][None, :, :], enc)
    o_ref[...] = acc + enc


@jax.jit
def kernel(x, track_ids, time_pe, track_pe):
    S, B, D = x.shape
    ids3 = track_ids.reshape(S, B, 1)
    grid = (S // S_BLK,)
    return pl.pallas_call(
        _body,
        grid=grid,
        in_specs=[
            pl.BlockSpec((S_BLK, B, D), lambda i: (i, 0, 0)),
            pl.BlockSpec((S_BLK, B, 1), lambda i: (i, 0, 0)),
            pl.BlockSpec((S_BLK, D), lambda i: (i, 0)),
            pl.BlockSpec((8, D), lambda i: (0, 0)),
        ],
        out_specs=pl.BlockSpec((S_BLK, B, D), lambda i: (i, 0, 0)),
        out_shape=jax.ShapeDtypeStruct((S, B, D), x.dtype),
        compiler_params=pltpu.CompilerParams(
            dimension_semantics=("arbitrary",),
        ),
    )(x, ids3, time_pe[:S], track_pe)
